# baseline (speedup 1.0000x reference)
import functools

import jax
import jax.numpy as jnp
from jax import lax
from jax.experimental import pallas as pl
from jax.experimental.pallas import tpu as pltpu
from jax.experimental.pallas import tpu_sc as plsc

NC = 2
NS = 16
NW = NC * NS
L = 16

V = 50000
F = 100000
FP = ((F + NW * L - 1) // (NW * L)) * (NW * L)
PER_W = FP // NW
NG = PER_W // L


def _bf16_hi(g):
    return plsc.bitcast(g & jnp.int32(-65536), jnp.float32)


def _bf16_lo(g):
    return plsc.bitcast(g << 16, jnp.float32)


@functools.partial(
    pl.kernel,
    out_type=jax.ShapeDtypeStruct((NW, 2 * L), jnp.float32),
    mesh=plsc.VectorSubcoreMesh(core_axis_name="c", subcore_axis_name="s"),
    compiler_params=pltpu.CompilerParams(needs_layout_passes=False),
    scratch_types=[
        pltpu.VMEM((2 * V,), jnp.int32),
        pltpu.VMEM((3 * PER_W,), jnp.int32),
        pltpu.VMEM((3 * PER_W,), jnp.float32),
        pltpu.VMEM((PER_W,), jnp.float32),
        pltpu.VMEM((2 * L,), jnp.float32),
        pltpu.SemaphoreType.DMA,
    ],
)
def _beam_gap_sc(tab_hbm, fc_hbm, pt_hbm, mk_hbm, out,
                 tab_v, fc_v, pt_v, mk_v, os_v, sem):
    wid = lax.axis_index("s") * NC + lax.axis_index("c")

    cps = (
        pltpu.async_copy(tab_hbm, tab_v, sem),
        pltpu.async_copy(fc_hbm.at[pl.ds(wid * 3 * PER_W, 3 * PER_W)], fc_v,
                         sem),
        pltpu.async_copy(pt_hbm.at[pl.ds(wid * 3 * PER_W, 3 * PER_W)], pt_v,
                         sem),
        pltpu.async_copy(mk_hbm.at[pl.ds(wid * PER_W, PER_W)], mk_v, sem),
    )
    for cp in cps:
        cp.wait()

    third = jnp.float32(1.0 / 3.0)
    half = jnp.float32(0.5)
    threehalf = jnp.float32(1.5)
    one = jnp.int32(1)
    i3 = lax.iota(jnp.int32, L) * jnp.int32(3)

    def body(g, carry):
        acc_s, acc_c = carry
        pa = i3 + g * jnp.int32(3 * L)
        pb = pa + one
        pc = pb + one
        ia = plsc.load_gather(fc_v, [pa]) << one
        ib = plsc.load_gather(fc_v, [pb]) << one
        ic = plsc.load_gather(fc_v, [pc]) << one
        ga = plsc.load_gather(tab_v, [ia])
        gb = plsc.load_gather(tab_v, [ib])
        gc = plsc.load_gather(tab_v, [ic])
        za = plsc.bitcast(plsc.load_gather(tab_v, [ia + one]), jnp.float32)
        zb = plsc.bitcast(plsc.load_gather(tab_v, [ib + one]), jnp.float32)
        zc = plsc.bitcast(plsc.load_gather(tab_v, [ic + one]), jnp.float32)
        mx = (_bf16_hi(ga) + _bf16_hi(gb) + _bf16_hi(gc)) * third
        my = (_bf16_lo(ga) + _bf16_lo(gb) + _bf16_lo(gc)) * third
        mz = (za + zb + zc) * third
        dx = plsc.load_gather(pt_v, [pa]) - mx
        dy = plsc.load_gather(pt_v, [pb]) - my
        dz = plsc.load_gather(pt_v, [pc]) - mz
        d2 = dx * dx + dy * dy + dz * dz
        d2m = jnp.maximum(d2, jnp.float32(1e-30))
        seed = jnp.int32(0x5F3759DF) - lax.shift_right_logical(
            plsc.bitcast(d2m, jnp.int32), one)
        y = plsc.bitcast(seed, jnp.float32)
        y = y * (threehalf - half * d2m * y * y)
        y = y * (threehalf - half * d2m * y * y)
        norm = d2 * y
        mk = mk_v[pl.ds(g * L, L)]
        return acc_s + norm * mk, acc_c + mk

    zeros = jnp.zeros((L,), jnp.float32)
    acc_s, acc_c = lax.fori_loop(0, NG, body, (zeros, zeros))

    os_v[pl.ds(0, L)] = acc_s
    os_v[pl.ds(L, L)] = acc_c
    pltpu.sync_copy(os_v, out.at[wid])


def kernel(points, mask, vertices, faces):
    pad = FP - F
    fc = jnp.pad(faces, ((0, pad), (0, 0))).reshape(-1)
    pt = jnp.pad(points, ((0, pad), (0, 0))).reshape(-1)
    mk = jnp.pad(mask, (0, pad)).astype(jnp.float32)

    xb = lax.bitcast_convert_type(
        vertices[:, 0].astype(jnp.bfloat16), jnp.uint16).astype(jnp.uint32)
    yb = lax.bitcast_convert_type(
        vertices[:, 1].astype(jnp.bfloat16), jnp.uint16).astype(jnp.uint32)
    xy = lax.bitcast_convert_type((xb << 16) | yb, jnp.int32)
    zb = lax.bitcast_convert_type(vertices[:, 2], jnp.int32)
    tab = jnp.stack([xy, zb], axis=1).reshape(-1)

    parts = _beam_gap_sc(tab, fc, pt, mk)
    sc = jnp.sum(parts.reshape(NW, 2, L), axis=(0, 2))
    l2 = 10.0 * (sc[0] / sc[1])
    return l2.astype(jnp.float32)
